# Initial kernel scaffold; baseline (speedup 1.0000x reference)
import jax
import jax.numpy as jnp
from jax.experimental import pallas as pl


def kernel(x, edge_index, batch, W0, b0, g0, be0, W1, b1, g1, be1, W2, b2, g2, be2, cW0, cb0, cW1, cb1, cW2, cb2):
    raise NotImplementedError("write your pallas kernel here")


# speedup vs baseline: 5.5940x; 5.5940x over previous
import functools

import jax
import jax.numpy as jnp
from jax import lax
from jax.experimental import pallas as pl
from jax.experimental.pallas import tpu as pltpu
from jax.experimental.pallas import tpu_sc as plsc

F32 = jnp.float32

_NC = 2
_NS = 16
_CHUNK = 80

_N = 10000
_RPT = 624
_RTAIL = _N - _RPT * _NS
_E = 320000
_G = 64
_NB = 5
_BR = _N // _NB


def _init_acc(zeros_hbm, acc, s):
    pltpu.sync_copy(zeros_hbm, acc.at[pl.ds(s * _RPT, _RPT)])

    @pl.when(s == _NS - 1)
    def _():
        pltpu.sync_copy(zeros_hbm.at[pl.ds(0, _RTAIL)],
                        acc.at[pl.ds(_RPT * _NS, _RTAIL)])


def _write_out(acc, out_hbm, c, s):
    r0 = s * _RPT
    pltpu.sync_copy(acc.at[pl.ds(r0, _RPT)],
                    out_hbm.at[c, pl.ds(r0, _RPT)])

    @pl.when(s == _NS - 1)
    def _():
        pltpu.sync_copy(acc.at[pl.ds(_RPT * _NS, _RTAIL)],
                        out_hbm.at[c, pl.ds(_RPT * _NS, _RTAIL)])


def _sc_degree(dst32, zeros_deg, ones_chunk):
    nchunks = dst32.shape[1]
    mesh = plsc.VectorSubcoreMesh(core_axis_name="c", subcore_axis_name="s")

    @functools.partial(
        pl.kernel,
        out_type=jax.ShapeDtypeStruct((_NC, _N, 128), F32),
        mesh=mesh,
        scratch_types=[
            pltpu.VMEM((nchunks, _CHUNK), jnp.int32),
            pltpu.VMEM((_CHUNK, 128), F32),
            pltpu.VMEM_SHARED((_N, 128), F32),
        ],
    )
    def k(dst_hbm, zeros_hbm, ones_hbm, out_hbm, didx, ones_v, acc):
        c = lax.axis_index("c")
        s = lax.axis_index("s")
        w = c * _NS + s
        _init_acc(zeros_hbm, acc, s)
        pltpu.sync_copy(ones_hbm, ones_v)
        pltpu.sync_copy(dst_hbm.at[w], didx)
        plsc.subcore_barrier()

        @pl.loop(0, nchunks)
        def _(j):
            pltpu.sync_copy(ones_v, acc.at[didx.at[j]], add=True)

        plsc.subcore_barrier()
        _write_out(acc, out_hbm, c, s)

    return k(dst32, zeros_deg, ones_chunk)


def _sc_spmm(uq, src, dst, zeros_blk):
    nchunks = _E // (_NS * _CHUNK)
    mesh = plsc.VectorSubcoreMesh(core_axis_name="c", subcore_axis_name="s")

    @functools.partial(
        pl.kernel,
        out_type=jax.ShapeDtypeStruct((_NC, _N, 128), F32),
        mesh=mesh,
        scratch_types=[
            pltpu.VMEM((_CHUNK,), jnp.int32),
            pltpu.VMEM((_CHUNK,), jnp.int32),
            pltpu.VMEM((_CHUNK, 128), F32),
            pltpu.VMEM_SHARED((_N, 128), F32),
        ],
    )
    def k(ua_hbm, ub_hbm, src_hbm, dst_hbm, zeros_hbm,
          out_hbm, sidx, didx, rows, acc):
        c = lax.axis_index("c")
        s = lax.axis_index("s")
        _init_acc(zeros_hbm, acc, s)
        plsc.subcore_barrier()

        def accumulate(table):
            @pl.loop(0, nchunks)
            def _(j):
                e0 = (s * nchunks + j) * _CHUNK
                pltpu.sync_copy(src_hbm.at[pl.ds(e0, _CHUNK)], sidx)
                pltpu.sync_copy(dst_hbm.at[pl.ds(e0, _CHUNK)], didx)
                pltpu.sync_copy(table.at[sidx], rows)
                pltpu.sync_copy(rows, acc.at[didx], add=True)

        for cval in (0, 1):
            @pl.when(c == cval)
            def _():
                accumulate((ua_hbm, ub_hbm)[cval])

        plsc.subcore_barrier()
        _write_out(acc, out_hbm, c, s)

    return k(uq[0], uq[1], src, dst, zeros_blk)


def _deg_from_parts(dp):
    deg = dp[0, :, 0:1] + dp[1, :, 0:1] + 1.0
    return deg


def _tc_prep0(x, W0, degparts):
    def body(x_ref, w_ref, dp_ref, u0_ref, u1_ref, self_ref):
        deg = _deg_from_parts(dp_ref[...])
        dinv = lax.rsqrt(deg)
        hw = jnp.dot(x_ref[...], w_ref[...],
                     preferred_element_type=F32,
                     precision=lax.Precision.HIGHEST)
        u = hw * dinv
        u0_ref[...] = u[:, 0:128]
        u1_ref[...] = u[:, 128:256]
        self_ref[...] = hw * (1.0 / deg)

    h = W0.shape[1]
    uspec = pl.BlockSpec((_BR, 128), lambda i: (i, 0))
    ushape = jax.ShapeDtypeStruct((_N, 128), F32)
    out = pl.pallas_call(
        body,
        grid=(_NB,),
        in_specs=[
            pl.BlockSpec((_BR, x.shape[1]), lambda i: (i, 0)),
            pl.BlockSpec(W0.shape, lambda i: (0, 0)),
            pl.BlockSpec((2, _BR, 128), lambda i: (0, i, 0)),
        ],
        out_specs=[uspec, uspec,
                   pl.BlockSpec((_BR, h), lambda i: (i, 0))],
        out_shape=[ushape, ushape,
                   jax.ShapeDtypeStruct((_N, h), F32)],
    )(x, W0, degparts)
    return out[:2], out[2]


def _tc_stats(S, selfh, degparts):
    h = selfh.shape[1]

    def body(s_ref, self_ref, dp_ref, pre_ref, ps_ref, pq_ref):
        i = pl.program_id(0)
        deg = _deg_from_parts(dp_ref[...])
        dinv = lax.rsqrt(deg)
        s2 = s_ref[...]
        scat = jnp.concatenate([s2[0], s2[1]], axis=1)
        pre = dinv * scat + self_ref[...]
        pre_ref[...] = pre

        @pl.when(i == 0)
        def _():
            ps_ref[...] = jnp.zeros((8, h), F32)
            pq_ref[...] = jnp.zeros((8, h), F32)

        ps_ref[...] += pre.reshape(_BR // 8, 8, h).sum(axis=0)
        pq_ref[...] += (pre * pre).reshape(_BR // 8, 8, h).sum(axis=0)

    return pl.pallas_call(
        body,
        grid=(_NB,),
        in_specs=[
            pl.BlockSpec((2, _BR, 128), lambda i: (0, i, 0)),
            pl.BlockSpec((_BR, h), lambda i: (i, 0)),
            pl.BlockSpec((2, _BR, 128), lambda i: (0, i, 0)),
        ],
        out_specs=[
            pl.BlockSpec((_BR, h), lambda i: (i, 0)),
            pl.BlockSpec((8, h), lambda i: (0, 0)),
            pl.BlockSpec((8, h), lambda i: (0, 0)),
        ],
        out_shape=[
            jax.ShapeDtypeStruct((_N, h), F32),
            jax.ShapeDtypeStruct((8, h), F32),
            jax.ShapeDtypeStruct((8, h), F32),
        ],
    )(S, selfh, degparts)


def _tc_prep(pre, psum, psumsq, g, be, W, degparts):
    h_in = pre.shape[1]
    h_out = W.shape[1]

    def body(pre_ref, ps_ref, pq_ref, g_ref, be_ref, w_ref, dp_ref,
             u0_ref, u1_ref, self_ref):
        mean = jnp.sum(ps_ref[...], axis=0, keepdims=True) / _N
        var = jnp.sum(pq_ref[...], axis=0, keepdims=True) / _N - mean * mean
        inv = lax.rsqrt(var + 1e-5)
        hrelu = jnp.maximum(
            (pre_ref[...] - mean) * inv * g_ref[...] + be_ref[...], 0.0)
        deg = _deg_from_parts(dp_ref[...])
        dinv = lax.rsqrt(deg)
        hw = jnp.dot(hrelu, w_ref[...],
                     preferred_element_type=F32,
                     precision=lax.Precision.HIGHEST)
        u = hw * dinv
        u0_ref[...] = u[:, 0:128]
        u1_ref[...] = u[:, 128:256]
        self_ref[...] = hw * (1.0 / deg)

    uspec = pl.BlockSpec((_BR, 128), lambda i: (i, 0))
    ushape = jax.ShapeDtypeStruct((_N, 128), F32)
    out = pl.pallas_call(
        body,
        grid=(_NB,),
        in_specs=[
            pl.BlockSpec((_BR, h_in), lambda i: (i, 0)),
            pl.BlockSpec((8, h_in), lambda i: (0, 0)),
            pl.BlockSpec((8, h_in), lambda i: (0, 0)),
            pl.BlockSpec((1, h_in), lambda i: (0, 0)),
            pl.BlockSpec((1, h_in), lambda i: (0, 0)),
            pl.BlockSpec((h_in, h_out), lambda i: (0, 0)),
            pl.BlockSpec((2, _BR, 128), lambda i: (0, i, 0)),
        ],
        out_specs=[uspec, uspec,
                   pl.BlockSpec((_BR, h_out), lambda i: (i, 0))],
        out_shape=[ushape, ushape,
                   jax.ShapeDtypeStruct((_N, h_out), F32)],
    )(pre, psum, psumsq, g, be, W, degparts)
    return out[:2], out[2]


def _tc_final(pre, psum, psumsq, g, be, batch3, batchf,
              cW0, cb0, cW1, cb1, cW2, cb2):
    h = pre.shape[1]
    h1 = cW1.shape[1]
    c_out = cW2.shape[1]
    neg_inf = float("-inf")

    def body(pre_ref, ps_ref, pq_ref, g_ref, be_ref, br_ref, bc_ref,
             w0_ref, b0_ref, w1_ref, b1_ref, w2_ref, b2_ref,
             out_ref, msum_s, mmax_s, mcnt_s):
        i = pl.program_id(0)

        @pl.when(i == 0)
        def _():
            msum_s[...] = jnp.zeros((_G, h), F32)
            mmax_s[...] = jnp.full((_G, h), neg_inf, F32)
            mcnt_s[...] = jnp.zeros((_G, 128), F32)

        mean = jnp.sum(ps_ref[...], axis=0, keepdims=True) / _N
        var = jnp.sum(pq_ref[...], axis=0, keepdims=True) / _N - mean * mean
        inv = lax.rsqrt(var + 1e-5)
        hrelu = jnp.maximum(
            (pre_ref[...] - mean) * inv * g_ref[...] + be_ref[...], 0.0)

        brow = br_ref[...][0]
        oht = (lax.broadcasted_iota(jnp.int32, (_G, _BR), 0) == brow)
        oht = oht.astype(F32)
        msum_s[...] += jnp.dot(oht, hrelu, preferred_element_type=F32,
                               precision=lax.Precision.HIGHEST)
        mcnt_s[...] += jnp.dot(oht, jnp.ones((_BR, 128), F32),
                               preferred_element_type=F32,
                               precision=lax.Precision.HIGHEST)

        bcol = bc_ref[...]

        def maxbody(gid, _):
            mask = bcol == gid.astype(F32)
            rowmax = jnp.max(jnp.where(mask, hrelu, neg_inf), axis=0,
                             keepdims=True)
            cur = mmax_s[pl.ds(gid, 1), :]
            mmax_s[pl.ds(gid, 1), :] = jnp.maximum(cur, rowmax)
            return 0

        lax.fori_loop(0, _G, maxbody, 0)

        @pl.when(i == _NB - 1)
        def _():
            cnt = mcnt_s[...][:, 0:1]
            mean_p = msum_s[...] / jnp.maximum(cnt, 1.0)
            mx = jnp.where(cnt > 0.0, mmax_s[...], 0.0)
            z = jnp.concatenate([mean_p, mx], axis=1)
            z1 = jnp.maximum(jnp.dot(z, w0_ref[...],
                                     preferred_element_type=F32,
                                     precision=lax.Precision.HIGHEST)
                             + b0_ref[...], 0.0)
            z2 = jnp.maximum(jnp.dot(z1, w1_ref[...],
                                     preferred_element_type=F32,
                                     precision=lax.Precision.HIGHEST)
                             + b1_ref[...], 0.0)
            lg = jnp.dot(z2, w2_ref[...], preferred_element_type=F32,
                         precision=lax.Precision.HIGHEST) + b2_ref[...]
            m = jnp.max(lg, axis=1, keepdims=True)
            lse = jnp.log(jnp.sum(jnp.exp(lg - m), axis=1,
                                  keepdims=True)) + m
            out_ref[...] = lg - lse

    return pl.pallas_call(
        body,
        grid=(_NB,),
        in_specs=[
            pl.BlockSpec((_BR, h), lambda i: (i, 0)),
            pl.BlockSpec((8, h), lambda i: (0, 0)),
            pl.BlockSpec((8, h), lambda i: (0, 0)),
            pl.BlockSpec((1, h), lambda i: (0, 0)),
            pl.BlockSpec((1, h), lambda i: (0, 0)),
            pl.BlockSpec((1, 1, _BR), lambda i: (i, 0, 0)),
            pl.BlockSpec((_BR, 1), lambda i: (i, 0)),
            pl.BlockSpec((2 * h, h), lambda i: (0, 0)),
            pl.BlockSpec((1, h), lambda i: (0, 0)),
            pl.BlockSpec((h, h1), lambda i: (0, 0)),
            pl.BlockSpec((1, h1), lambda i: (0, 0)),
            pl.BlockSpec((h1, c_out), lambda i: (0, 0)),
            pl.BlockSpec((1, c_out), lambda i: (0, 0)),
        ],
        out_specs=pl.BlockSpec((_G, c_out), lambda i: (0, 0)),
        out_shape=jax.ShapeDtypeStruct((_G, c_out), F32),
        scratch_shapes=[
            pltpu.VMEM((_G, h), F32),
            pltpu.VMEM((_G, h), F32),
            pltpu.VMEM((_G, 128), F32),
        ],
    )(pre, psum, psumsq, g, be, batch3, batchf,
      cW0, cb0, cW1, cb1, cW2, cb2)


def kernel(x, edge_index, batch,
           W0, b0, g0, be0, W1, b1, g1, be1, W2, b2, g2, be2,
           cW0, cb0, cW1, cb1, cW2, cb2):
    del b0, b1, b2
    src = edge_index[0]
    dst = edge_index[1]
    dst32 = dst.reshape(_NC * _NS, -1, _CHUNK)
    zeros_blk = jnp.zeros((_RPT, 128), F32)
    ones_chunk = jnp.ones((_CHUNK, 128), F32)
    batch3 = batch.reshape(_NB, 1, _BR)
    batchf = batch.astype(F32).reshape(_N, 1)
    g0r, be0r = g0.reshape(1, -1), be0.reshape(1, -1)
    g1r, be1r = g1.reshape(1, -1), be1.reshape(1, -1)
    g2r, be2r = g2.reshape(1, -1), be2.reshape(1, -1)
    cb0r, cb1r = cb0.reshape(1, -1), cb1.reshape(1, -1)
    c_real = cW2.shape[1]
    c_pad = 256 - c_real
    cW2p = jnp.concatenate([cW2, jnp.zeros((cW2.shape[0], c_pad), F32)], 1)
    cb2r = jnp.concatenate([cb2, jnp.full((c_pad,), -1e30, F32)],
                           0).reshape(1, -1)

    degparts = _sc_degree(dst32, zeros_blk, ones_chunk)

    uq, selfh = _tc_prep0(x, W0, degparts)
    S = _sc_spmm(uq, src, dst, zeros_blk)
    pre, ps, pq = _tc_stats(S, selfh, degparts)

    uq, selfh = _tc_prep(pre, ps, pq, g0r, be0r, W1, degparts)
    S = _sc_spmm(uq, src, dst, zeros_blk)
    pre, ps, pq = _tc_stats(S, selfh, degparts)

    uq, selfh = _tc_prep(pre, ps, pq, g1r, be1r, W2, degparts)
    S = _sc_spmm(uq, src, dst, zeros_blk)
    pre, ps, pq = _tc_stats(S, selfh, degparts)

    out = _tc_final(pre, ps, pq, g2r, be2r, batch3, batchf,
                    cW0, cb0r, cW1, cb1r, cW2p, cb2r)
    return out[:, :c_real]


# speedup vs baseline: 8.3820x; 1.4984x over previous
import functools

import jax
import jax.numpy as jnp
from jax import lax
from jax.experimental import pallas as pl
from jax.experimental.pallas import tpu as pltpu
from jax.experimental.pallas import tpu_sc as plsc

F32 = jnp.float32

_NC = 2
_NS = 16
_CHUNK = 80

_N = 10000
_RPT = 624
_RTAIL = _N - _RPT * _NS
_E = 320000
_G = 64
_NB = 5
_BR = _N // _NB


def _init_acc(zeros_hbm, acc, s):
    pltpu.sync_copy(zeros_hbm, acc.at[pl.ds(s * _RPT, _RPT)])

    @pl.when(s == _NS - 1)
    def _():
        pltpu.sync_copy(zeros_hbm.at[pl.ds(0, _RTAIL)],
                        acc.at[pl.ds(_RPT * _NS, _RTAIL)])


def _write_out(acc, out_hbm, c, s):
    r0 = s * _RPT
    pltpu.sync_copy(acc.at[pl.ds(r0, _RPT)],
                    out_hbm.at[c, pl.ds(r0, _RPT)])

    @pl.when(s == _NS - 1)
    def _():
        pltpu.sync_copy(acc.at[pl.ds(_RPT * _NS, _RTAIL)],
                        out_hbm.at[c, pl.ds(_RPT * _NS, _RTAIL)])


def _sc_degree(dst32, zeros_deg, ones_chunk):
    nchunks = dst32.shape[1]
    mesh = plsc.VectorSubcoreMesh(core_axis_name="c", subcore_axis_name="s")

    @functools.partial(
        pl.kernel,
        out_type=jax.ShapeDtypeStruct((_NC, _N, 128), F32),
        mesh=mesh,
        scratch_types=[
            pltpu.VMEM((nchunks, _CHUNK), jnp.int32),
            pltpu.VMEM((_CHUNK, 128), F32),
            pltpu.VMEM_SHARED((_N, 128), F32),
        ],
    )
    def k(dst_hbm, zeros_hbm, ones_hbm, out_hbm, didx, ones_v, acc):
        c = lax.axis_index("c")
        s = lax.axis_index("s")
        w = c * _NS + s
        _init_acc(zeros_hbm, acc, s)
        pltpu.sync_copy(ones_hbm, ones_v)
        pltpu.sync_copy(dst_hbm.at[w], didx)
        plsc.subcore_barrier()

        @pl.loop(0, nchunks)
        def _(j):
            pltpu.sync_copy(ones_v, acc.at[didx.at[j]], add=True)

        plsc.subcore_barrier()
        _write_out(acc, out_hbm, c, s)

    return k(dst32, zeros_deg, ones_chunk)


def _sc_spmm(uq, eidx4, zeros_blk):
    npairs = _E // (_NS * 2 * _CHUNK)
    mesh = plsc.VectorSubcoreMesh(core_axis_name="c", subcore_axis_name="s")

    @functools.partial(
        pl.kernel,
        out_type=jax.ShapeDtypeStruct((_NC, _N, 128), F32),
        mesh=mesh,
        scratch_types=[
            pltpu.VMEM((4, _CHUNK), jnp.int32),
            pltpu.VMEM((_CHUNK, 128), F32),
            pltpu.VMEM((_CHUNK, 128), F32),
            pltpu.VMEM_SHARED((_N, 128), F32),
            pltpu.SemaphoreType.DMA,
            pltpu.SemaphoreType.DMA,
        ],
    )
    def k(ua_hbm, ub_hbm, ei_hbm, zeros_hbm,
          out_hbm, idxb, rows0, rows1, acc, sem0, sem1):
        c = lax.axis_index("c")
        s = lax.axis_index("s")
        _init_acc(zeros_hbm, acc, s)
        plsc.subcore_barrier()

        def accumulate(table):
            @pl.loop(0, npairs)
            def _(p):
                pltpu.sync_copy(ei_hbm.at[s, p], idxb)
                h0 = pltpu.async_copy(table.at[idxb.at[0]], rows0, sem0)
                h1 = pltpu.async_copy(table.at[idxb.at[1]], rows1, sem1)
                h0.wait()
                pltpu.sync_copy(rows0, acc.at[idxb.at[2]], add=True)
                h1.wait()
                pltpu.sync_copy(rows1, acc.at[idxb.at[3]], add=True)

        for cval in (0, 1):
            @pl.when(c == cval)
            def _():
                accumulate((ua_hbm, ub_hbm)[cval])

        plsc.subcore_barrier()
        _write_out(acc, out_hbm, c, s)

    return k(uq[0], uq[1], eidx4, zeros_blk)


def _deg_from_parts(dp):
    deg = dp[0, :, 0:1] + dp[1, :, 0:1] + 1.0
    return deg


def _tc_prep0(x, W0, degparts):
    def body(x_ref, w_ref, dp_ref, u0_ref, u1_ref, self_ref):
        deg = _deg_from_parts(dp_ref[...])
        dinv = lax.rsqrt(deg)
        hw = jnp.dot(x_ref[...], w_ref[...],
                     preferred_element_type=F32,
                     precision=lax.Precision.HIGHEST)
        u = hw * dinv
        u0_ref[...] = u[:, 0:128]
        u1_ref[...] = u[:, 128:256]
        self_ref[...] = hw * (1.0 / deg)

    h = W0.shape[1]
    uspec = pl.BlockSpec((_BR, 128), lambda i: (i, 0))
    ushape = jax.ShapeDtypeStruct((_N, 128), F32)
    out = pl.pallas_call(
        body,
        grid=(_NB,),
        in_specs=[
            pl.BlockSpec((_BR, x.shape[1]), lambda i: (i, 0)),
            pl.BlockSpec(W0.shape, lambda i: (0, 0)),
            pl.BlockSpec((2, _BR, 128), lambda i: (0, i, 0)),
        ],
        out_specs=[uspec, uspec,
                   pl.BlockSpec((_BR, h), lambda i: (i, 0))],
        out_shape=[ushape, ushape,
                   jax.ShapeDtypeStruct((_N, h), F32)],
    )(x, W0, degparts)
    return out[:2], out[2]


def _tc_stats(S, selfh, degparts):
    h = selfh.shape[1]

    def body(s_ref, self_ref, dp_ref, pre_ref, ps_ref, pq_ref):
        i = pl.program_id(0)
        deg = _deg_from_parts(dp_ref[...])
        dinv = lax.rsqrt(deg)
        s2 = s_ref[...]
        scat = jnp.concatenate([s2[0], s2[1]], axis=1)
        pre = dinv * scat + self_ref[...]
        pre_ref[...] = pre

        @pl.when(i == 0)
        def _():
            ps_ref[...] = jnp.zeros((8, h), F32)
            pq_ref[...] = jnp.zeros((8, h), F32)

        ps_ref[...] += pre.reshape(_BR // 8, 8, h).sum(axis=0)
        pq_ref[...] += (pre * pre).reshape(_BR // 8, 8, h).sum(axis=0)

    return pl.pallas_call(
        body,
        grid=(_NB,),
        in_specs=[
            pl.BlockSpec((2, _BR, 128), lambda i: (0, i, 0)),
            pl.BlockSpec((_BR, h), lambda i: (i, 0)),
            pl.BlockSpec((2, _BR, 128), lambda i: (0, i, 0)),
        ],
        out_specs=[
            pl.BlockSpec((_BR, h), lambda i: (i, 0)),
            pl.BlockSpec((8, h), lambda i: (0, 0)),
            pl.BlockSpec((8, h), lambda i: (0, 0)),
        ],
        out_shape=[
            jax.ShapeDtypeStruct((_N, h), F32),
            jax.ShapeDtypeStruct((8, h), F32),
            jax.ShapeDtypeStruct((8, h), F32),
        ],
    )(S, selfh, degparts)


def _tc_prep(pre, psum, psumsq, g, be, W, degparts):
    h_in = pre.shape[1]
    h_out = W.shape[1]

    def body(pre_ref, ps_ref, pq_ref, g_ref, be_ref, w_ref, dp_ref,
             u0_ref, u1_ref, self_ref):
        mean = jnp.sum(ps_ref[...], axis=0, keepdims=True) / _N
        var = jnp.sum(pq_ref[...], axis=0, keepdims=True) / _N - mean * mean
        inv = lax.rsqrt(var + 1e-5)
        hrelu = jnp.maximum(
            (pre_ref[...] - mean) * inv * g_ref[...] + be_ref[...], 0.0)
        deg = _deg_from_parts(dp_ref[...])
        dinv = lax.rsqrt(deg)
        hw = jnp.dot(hrelu, w_ref[...],
                     preferred_element_type=F32,
                     precision=lax.Precision.HIGHEST)
        u = hw * dinv
        u0_ref[...] = u[:, 0:128]
        u1_ref[...] = u[:, 128:256]
        self_ref[...] = hw * (1.0 / deg)

    uspec = pl.BlockSpec((_BR, 128), lambda i: (i, 0))
    ushape = jax.ShapeDtypeStruct((_N, 128), F32)
    out = pl.pallas_call(
        body,
        grid=(_NB,),
        in_specs=[
            pl.BlockSpec((_BR, h_in), lambda i: (i, 0)),
            pl.BlockSpec((8, h_in), lambda i: (0, 0)),
            pl.BlockSpec((8, h_in), lambda i: (0, 0)),
            pl.BlockSpec((1, h_in), lambda i: (0, 0)),
            pl.BlockSpec((1, h_in), lambda i: (0, 0)),
            pl.BlockSpec((h_in, h_out), lambda i: (0, 0)),
            pl.BlockSpec((2, _BR, 128), lambda i: (0, i, 0)),
        ],
        out_specs=[uspec, uspec,
                   pl.BlockSpec((_BR, h_out), lambda i: (i, 0))],
        out_shape=[ushape, ushape,
                   jax.ShapeDtypeStruct((_N, h_out), F32)],
    )(pre, psum, psumsq, g, be, W, degparts)
    return out[:2], out[2]


def _tc_final(pre, psum, psumsq, g, be, batch3, batchf,
              cW0, cb0, cW1, cb1, cW2, cb2):
    h = pre.shape[1]
    h1 = cW1.shape[1]
    c_out = cW2.shape[1]
    neg_inf = float("-inf")

    def body(pre_ref, ps_ref, pq_ref, g_ref, be_ref, br_ref, bc_ref,
             w0_ref, b0_ref, w1_ref, b1_ref, w2_ref, b2_ref,
             out_ref, msum_s, mmax_s, mcnt_s):
        i = pl.program_id(0)

        @pl.when(i == 0)
        def _():
            msum_s[...] = jnp.zeros((_G, h), F32)
            mmax_s[...] = jnp.full((_G, h), neg_inf, F32)
            mcnt_s[...] = jnp.zeros((_G, 128), F32)

        mean = jnp.sum(ps_ref[...], axis=0, keepdims=True) / _N
        var = jnp.sum(pq_ref[...], axis=0, keepdims=True) / _N - mean * mean
        inv = lax.rsqrt(var + 1e-5)
        hrelu = jnp.maximum(
            (pre_ref[...] - mean) * inv * g_ref[...] + be_ref[...], 0.0)

        brow = br_ref[...][0]
        oht = (lax.broadcasted_iota(jnp.int32, (_G, _BR), 0) == brow)
        oht = oht.astype(F32)
        msum_s[...] += jnp.dot(oht, hrelu, preferred_element_type=F32,
                               precision=lax.Precision.HIGHEST)
        mcnt_s[...] += jnp.dot(oht, jnp.ones((_BR, 128), F32),
                               preferred_element_type=F32,
                               precision=lax.Precision.HIGHEST)

        bcol = bc_ref[...]

        def maxbody(gid, _):
            mask = bcol == gid.astype(F32)
            rowmax = jnp.max(jnp.where(mask, hrelu, neg_inf), axis=0,
                             keepdims=True)
            cur = mmax_s[pl.ds(gid, 1), :]
            mmax_s[pl.ds(gid, 1), :] = jnp.maximum(cur, rowmax)
            return 0

        lax.fori_loop(0, _G, maxbody, 0)

        @pl.when(i == _NB - 1)
        def _():
            cnt = mcnt_s[...][:, 0:1]
            mean_p = msum_s[...] / jnp.maximum(cnt, 1.0)
            mx = jnp.where(cnt > 0.0, mmax_s[...], 0.0)
            z = jnp.concatenate([mean_p, mx], axis=1)
            z1 = jnp.maximum(jnp.dot(z, w0_ref[...],
                                     preferred_element_type=F32,
                                     precision=lax.Precision.HIGHEST)
                             + b0_ref[...], 0.0)
            z2 = jnp.maximum(jnp.dot(z1, w1_ref[...],
                                     preferred_element_type=F32,
                                     precision=lax.Precision.HIGHEST)
                             + b1_ref[...], 0.0)
            lg = jnp.dot(z2, w2_ref[...], preferred_element_type=F32,
                         precision=lax.Precision.HIGHEST) + b2_ref[...]
            m = jnp.max(lg, axis=1, keepdims=True)
            lse = jnp.log(jnp.sum(jnp.exp(lg - m), axis=1,
                                  keepdims=True)) + m
            out_ref[...] = lg - lse

    return pl.pallas_call(
        body,
        grid=(_NB,),
        in_specs=[
            pl.BlockSpec((_BR, h), lambda i: (i, 0)),
            pl.BlockSpec((8, h), lambda i: (0, 0)),
            pl.BlockSpec((8, h), lambda i: (0, 0)),
            pl.BlockSpec((1, h), lambda i: (0, 0)),
            pl.BlockSpec((1, h), lambda i: (0, 0)),
            pl.BlockSpec((1, 1, _BR), lambda i: (i, 0, 0)),
            pl.BlockSpec((_BR, 1), lambda i: (i, 0)),
            pl.BlockSpec((2 * h, h), lambda i: (0, 0)),
            pl.BlockSpec((1, h), lambda i: (0, 0)),
            pl.BlockSpec((h, h1), lambda i: (0, 0)),
            pl.BlockSpec((1, h1), lambda i: (0, 0)),
            pl.BlockSpec((h1, c_out), lambda i: (0, 0)),
            pl.BlockSpec((1, c_out), lambda i: (0, 0)),
        ],
        out_specs=pl.BlockSpec((_G, c_out), lambda i: (0, 0)),
        out_shape=jax.ShapeDtypeStruct((_G, c_out), F32),
        scratch_shapes=[
            pltpu.VMEM((_G, h), F32),
            pltpu.VMEM((_G, h), F32),
            pltpu.VMEM((_G, 128), F32),
        ],
    )(pre, psum, psumsq, g, be, batch3, batchf,
      cW0, cb0, cW1, cb1, cW2, cb2)


def kernel(x, edge_index, batch,
           W0, b0, g0, be0, W1, b1, g1, be1, W2, b2, g2, be2,
           cW0, cb0, cW1, cb1, cW2, cb2):
    del b0, b1, b2
    src = edge_index[0]
    dst = edge_index[1]
    dst32 = dst.reshape(_NC * _NS, -1, _CHUNK)
    eidx4 = jnp.concatenate([src.reshape(_NS, -1, 2, _CHUNK),
                             dst.reshape(_NS, -1, 2, _CHUNK)], axis=2)
    zeros_blk = jnp.zeros((_RPT, 128), F32)
    ones_chunk = jnp.ones((_CHUNK, 128), F32)
    batch3 = batch.reshape(_NB, 1, _BR)
    batchf = batch.astype(F32).reshape(_N, 1)
    g0r, be0r = g0.reshape(1, -1), be0.reshape(1, -1)
    g1r, be1r = g1.reshape(1, -1), be1.reshape(1, -1)
    g2r, be2r = g2.reshape(1, -1), be2.reshape(1, -1)
    cb0r, cb1r = cb0.reshape(1, -1), cb1.reshape(1, -1)
    c_real = cW2.shape[1]
    c_pad = 256 - c_real
    cW2p = jnp.concatenate([cW2, jnp.zeros((cW2.shape[0], c_pad), F32)], 1)
    cb2r = jnp.concatenate([cb2, jnp.full((c_pad,), -1e30, F32)],
                           0).reshape(1, -1)

    degparts = _sc_degree(dst32, zeros_blk, ones_chunk)

    uq, selfh = _tc_prep0(x, W0, degparts)
    S = _sc_spmm(uq, eidx4, zeros_blk)
    pre, ps, pq = _tc_stats(S, selfh, degparts)

    uq, selfh = _tc_prep(pre, ps, pq, g0r, be0r, W1, degparts)
    S = _sc_spmm(uq, eidx4, zeros_blk)
    pre, ps, pq = _tc_stats(S, selfh, degparts)

    uq, selfh = _tc_prep(pre, ps, pq, g1r, be1r, W2, degparts)
    S = _sc_spmm(uq, eidx4, zeros_blk)
    pre, ps, pq = _tc_stats(S, selfh, degparts)

    out = _tc_final(pre, ps, pq, g2r, be2r, batch3, batchf,
                    cW0, cb0r, cW1, cb1r, cW2p, cb2r)
    return out[:, :c_real]


# speedup vs baseline: 10.9021x; 1.3007x over previous
import functools

import jax
import jax.numpy as jnp
from jax import lax
from jax.experimental import pallas as pl
from jax.experimental.pallas import tpu as pltpu
from jax.experimental.pallas import tpu_sc as plsc

F32 = jnp.float32

_NC = 2
_NS = 16
_CHUNK = 80

_N = 10000
_RPT = 624
_RTAIL = _N - _RPT * _NS
_E = 320000
_G = 64
_NB = 5
_BR = _N // _NB


def _init_acc(zeros_hbm, acc, s):
    pltpu.sync_copy(zeros_hbm, acc.at[pl.ds(s * _RPT, _RPT)])

    @pl.when(s == _NS - 1)
    def _():
        pltpu.sync_copy(zeros_hbm.at[pl.ds(0, _RTAIL)],
                        acc.at[pl.ds(_RPT * _NS, _RTAIL)])


def _write_out(acc, out_hbm, c, s):
    r0 = s * _RPT
    pltpu.sync_copy(acc.at[pl.ds(r0, _RPT)],
                    out_hbm.at[c, pl.ds(r0, _RPT)])

    @pl.when(s == _NS - 1)
    def _():
        pltpu.sync_copy(acc.at[pl.ds(_RPT * _NS, _RTAIL)],
                        out_hbm.at[c, pl.ds(_RPT * _NS, _RTAIL)])


def _sc_degree(dst32, zeros_deg, ones_chunk):
    nchunks = dst32.shape[1]
    mesh = plsc.VectorSubcoreMesh(core_axis_name="c", subcore_axis_name="s")

    @functools.partial(
        pl.kernel,
        out_type=jax.ShapeDtypeStruct((_NC, _N, 128), F32),
        mesh=mesh,
        scratch_types=[
            pltpu.VMEM((nchunks, _CHUNK), jnp.int32),
            pltpu.VMEM((_CHUNK, 128), F32),
            pltpu.VMEM_SHARED((_N, 128), F32),
        ],
    )
    def k(dst_hbm, zeros_hbm, ones_hbm, out_hbm, didx, ones_v, acc):
        c = lax.axis_index("c")
        s = lax.axis_index("s")
        w = c * _NS + s
        _init_acc(zeros_hbm, acc, s)
        pltpu.sync_copy(ones_hbm, ones_v)
        pltpu.sync_copy(dst_hbm.at[w], didx)
        plsc.subcore_barrier()

        @pl.loop(0, nchunks)
        def _(j):
            pltpu.sync_copy(ones_v, acc.at[didx.at[j]], add=True)

        plsc.subcore_barrier()
        _write_out(acc, out_hbm, c, s)

    return k(dst32, zeros_deg, ones_chunk)


_SB = 10


def _sc_spmm(uq, eidx4, zeros_blk):
    nsb = _E // (_NS * _SB * _CHUNK)
    mesh = plsc.VectorSubcoreMesh(core_axis_name="c", subcore_axis_name="s")

    @functools.partial(
        pl.kernel,
        out_type=jax.ShapeDtypeStruct((_NC, _N, 128), F32),
        mesh=mesh,
        scratch_types=[
            pltpu.VMEM((2 * _SB, _CHUNK), jnp.int32),
            pltpu.VMEM((_CHUNK, 128), F32),
            pltpu.VMEM((_CHUNK, 128), F32),
            pltpu.VMEM_SHARED((_N, 128), F32),
            pltpu.SemaphoreType.DMA,
            pltpu.SemaphoreType.DMA,
        ],
    )
    def k(ua_hbm, ub_hbm, ei_hbm, zeros_hbm,
          out_hbm, idxb, rows0, rows1, acc, sem0, sem1):
        c = lax.axis_index("c")
        s = lax.axis_index("s")
        rows = (rows0, rows1)
        sems = (sem0, sem1)
        _init_acc(zeros_hbm, acc, s)
        plsc.subcore_barrier()

        def accumulate(table):
            @pl.loop(0, nsb)
            def _(b):
                pltpu.sync_copy(ei_hbm.at[s, b], idxb)
                hs = pltpu.async_copy(table.at[idxb.at[0]], rows[0], sems[0])
                for j in range(_SB):
                    if j + 1 < _SB:
                        hn = pltpu.async_copy(table.at[idxb.at[j + 1]],
                                              rows[(j + 1) % 2],
                                              sems[(j + 1) % 2])
                    hs.wait()
                    pltpu.sync_copy(rows[j % 2], acc.at[idxb.at[_SB + j]],
                                    add=True)
                    if j + 1 < _SB:
                        hs = hn

        for cval in (0, 1):
            @pl.when(c == cval)
            def _():
                accumulate((ua_hbm, ub_hbm)[cval])

        plsc.subcore_barrier()
        _write_out(acc, out_hbm, c, s)

    return k(uq[0], uq[1], eidx4, zeros_blk)


def _deg_from_parts(dp):
    deg = dp[0, :, 0:1] + dp[1, :, 0:1] + 1.0
    return deg


def _tc_prep0(x, W0, degparts):
    def body(x_ref, w_ref, dp_ref, u0_ref, u1_ref, self_ref):
        deg = _deg_from_parts(dp_ref[...])
        dinv = lax.rsqrt(deg)
        hw = jnp.dot(x_ref[...], w_ref[...],
                     preferred_element_type=F32,
                     precision=lax.Precision.HIGHEST)
        u = hw * dinv
        u0_ref[...] = u[:, 0:128]
        u1_ref[...] = u[:, 128:256]
        self_ref[...] = hw * (1.0 / deg)

    h = W0.shape[1]
    uspec = pl.BlockSpec((_BR, 128), lambda i: (i, 0))
    ushape = jax.ShapeDtypeStruct((_N, 128), F32)
    out = pl.pallas_call(
        body,
        grid=(_NB,),
        in_specs=[
            pl.BlockSpec((_BR, x.shape[1]), lambda i: (i, 0)),
            pl.BlockSpec(W0.shape, lambda i: (0, 0)),
            pl.BlockSpec((2, _BR, 128), lambda i: (0, i, 0)),
        ],
        out_specs=[uspec, uspec,
                   pl.BlockSpec((_BR, h), lambda i: (i, 0))],
        out_shape=[ushape, ushape,
                   jax.ShapeDtypeStruct((_N, h), F32)],
    )(x, W0, degparts)
    return out[:2], out[2]


def _tc_stats(S, selfh, degparts):
    h = selfh.shape[1]

    def body(s_ref, self_ref, dp_ref, pre_ref, ps_ref, pq_ref):
        i = pl.program_id(0)
        deg = _deg_from_parts(dp_ref[...])
        dinv = lax.rsqrt(deg)
        s2 = s_ref[...]
        scat = jnp.concatenate([s2[0], s2[1]], axis=1)
        pre = dinv * scat + self_ref[...]
        pre_ref[...] = pre

        @pl.when(i == 0)
        def _():
            ps_ref[...] = jnp.zeros((8, h), F32)
            pq_ref[...] = jnp.zeros((8, h), F32)

        ps_ref[...] += pre.reshape(_BR // 8, 8, h).sum(axis=0)
        pq_ref[...] += (pre * pre).reshape(_BR // 8, 8, h).sum(axis=0)

    return pl.pallas_call(
        body,
        grid=(_NB,),
        in_specs=[
            pl.BlockSpec((2, _BR, 128), lambda i: (0, i, 0)),
            pl.BlockSpec((_BR, h), lambda i: (i, 0)),
            pl.BlockSpec((2, _BR, 128), lambda i: (0, i, 0)),
        ],
        out_specs=[
            pl.BlockSpec((_BR, h), lambda i: (i, 0)),
            pl.BlockSpec((8, h), lambda i: (0, 0)),
            pl.BlockSpec((8, h), lambda i: (0, 0)),
        ],
        out_shape=[
            jax.ShapeDtypeStruct((_N, h), F32),
            jax.ShapeDtypeStruct((8, h), F32),
            jax.ShapeDtypeStruct((8, h), F32),
        ],
    )(S, selfh, degparts)


def _tc_prep(pre, psum, psumsq, g, be, W, degparts):
    h_in = pre.shape[1]
    h_out = W.shape[1]

    def body(pre_ref, ps_ref, pq_ref, g_ref, be_ref, w_ref, dp_ref,
             u0_ref, u1_ref, self_ref):
        mean = jnp.sum(ps_ref[...], axis=0, keepdims=True) / _N
        var = jnp.sum(pq_ref[...], axis=0, keepdims=True) / _N - mean * mean
        inv = lax.rsqrt(var + 1e-5)
        hrelu = jnp.maximum(
            (pre_ref[...] - mean) * inv * g_ref[...] + be_ref[...], 0.0)
        deg = _deg_from_parts(dp_ref[...])
        dinv = lax.rsqrt(deg)
        hw = jnp.dot(hrelu, w_ref[...],
                     preferred_element_type=F32,
                     precision=lax.Precision.HIGHEST)
        u = hw * dinv
        u0_ref[...] = u[:, 0:128]
        u1_ref[...] = u[:, 128:256]
        self_ref[...] = hw * (1.0 / deg)

    uspec = pl.BlockSpec((_BR, 128), lambda i: (i, 0))
    ushape = jax.ShapeDtypeStruct((_N, 128), F32)
    out = pl.pallas_call(
        body,
        grid=(_NB,),
        in_specs=[
            pl.BlockSpec((_BR, h_in), lambda i: (i, 0)),
            pl.BlockSpec((8, h_in), lambda i: (0, 0)),
            pl.BlockSpec((8, h_in), lambda i: (0, 0)),
            pl.BlockSpec((1, h_in), lambda i: (0, 0)),
            pl.BlockSpec((1, h_in), lambda i: (0, 0)),
            pl.BlockSpec((h_in, h_out), lambda i: (0, 0)),
            pl.BlockSpec((2, _BR, 128), lambda i: (0, i, 0)),
        ],
        out_specs=[uspec, uspec,
                   pl.BlockSpec((_BR, h_out), lambda i: (i, 0))],
        out_shape=[ushape, ushape,
                   jax.ShapeDtypeStruct((_N, h_out), F32)],
    )(pre, psum, psumsq, g, be, W, degparts)
    return out[:2], out[2]


def _tc_final(pre, psum, psumsq, g, be, batch3, batchf,
              cW0, cb0, cW1, cb1, cW2, cb2):
    h = pre.shape[1]
    h1 = cW1.shape[1]
    c_out = cW2.shape[1]
    neg_inf = float("-inf")

    def body(pre_ref, ps_ref, pq_ref, g_ref, be_ref, br_ref, bc_ref,
             w0_ref, b0_ref, w1_ref, b1_ref, w2_ref, b2_ref,
             out_ref, msum_s, mmax_s, mcnt_s):
        i = pl.program_id(0)

        @pl.when(i == 0)
        def _():
            msum_s[...] = jnp.zeros((_G, h), F32)
            mmax_s[...] = jnp.full((_G, h), neg_inf, F32)
            mcnt_s[...] = jnp.zeros((_G, 128), F32)

        mean = jnp.sum(ps_ref[...], axis=0, keepdims=True) / _N
        var = jnp.sum(pq_ref[...], axis=0, keepdims=True) / _N - mean * mean
        inv = lax.rsqrt(var + 1e-5)
        hrelu = jnp.maximum(
            (pre_ref[...] - mean) * inv * g_ref[...] + be_ref[...], 0.0)

        brow = br_ref[...][0]
        oht = (lax.broadcasted_iota(jnp.int32, (_G, _BR), 0) == brow)
        oht = oht.astype(F32)
        msum_s[...] += jnp.dot(oht, hrelu, preferred_element_type=F32,
                               precision=lax.Precision.HIGHEST)
        mcnt_s[...] += jnp.dot(oht, jnp.ones((_BR, 128), F32),
                               preferred_element_type=F32,
                               precision=lax.Precision.HIGHEST)

        bcol = bc_ref[...]

        def maxbody(gid, _):
            mask = bcol == gid.astype(F32)
            rowmax = jnp.max(jnp.where(mask, hrelu, neg_inf), axis=0,
                             keepdims=True)
            cur = mmax_s[pl.ds(gid, 1), :]
            mmax_s[pl.ds(gid, 1), :] = jnp.maximum(cur, rowmax)
            return 0

        lax.fori_loop(0, _G, maxbody, 0)

        @pl.when(i == _NB - 1)
        def _():
            cnt = mcnt_s[...][:, 0:1]
            mean_p = msum_s[...] / jnp.maximum(cnt, 1.0)
            mx = jnp.where(cnt > 0.0, mmax_s[...], 0.0)
            z = jnp.concatenate([mean_p, mx], axis=1)
            z1 = jnp.maximum(jnp.dot(z, w0_ref[...],
                                     preferred_element_type=F32,
                                     precision=lax.Precision.HIGHEST)
                             + b0_ref[...], 0.0)
            z2 = jnp.maximum(jnp.dot(z1, w1_ref[...],
                                     preferred_element_type=F32,
                                     precision=lax.Precision.HIGHEST)
                             + b1_ref[...], 0.0)
            lg = jnp.dot(z2, w2_ref[...], preferred_element_type=F32,
                         precision=lax.Precision.HIGHEST) + b2_ref[...]
            m = jnp.max(lg, axis=1, keepdims=True)
            lse = jnp.log(jnp.sum(jnp.exp(lg - m), axis=1,
                                  keepdims=True)) + m
            out_ref[...] = lg - lse

    return pl.pallas_call(
        body,
        grid=(_NB,),
        in_specs=[
            pl.BlockSpec((_BR, h), lambda i: (i, 0)),
            pl.BlockSpec((8, h), lambda i: (0, 0)),
            pl.BlockSpec((8, h), lambda i: (0, 0)),
            pl.BlockSpec((1, h), lambda i: (0, 0)),
            pl.BlockSpec((1, h), lambda i: (0, 0)),
            pl.BlockSpec((1, 1, _BR), lambda i: (i, 0, 0)),
            pl.BlockSpec((_BR, 1), lambda i: (i, 0)),
            pl.BlockSpec((2 * h, h), lambda i: (0, 0)),
            pl.BlockSpec((1, h), lambda i: (0, 0)),
            pl.BlockSpec((h, h1), lambda i: (0, 0)),
            pl.BlockSpec((1, h1), lambda i: (0, 0)),
            pl.BlockSpec((h1, c_out), lambda i: (0, 0)),
            pl.BlockSpec((1, c_out), lambda i: (0, 0)),
        ],
        out_specs=pl.BlockSpec((_G, c_out), lambda i: (0, 0)),
        out_shape=jax.ShapeDtypeStruct((_G, c_out), F32),
        scratch_shapes=[
            pltpu.VMEM((_G, h), F32),
            pltpu.VMEM((_G, h), F32),
            pltpu.VMEM((_G, 128), F32),
        ],
    )(pre, psum, psumsq, g, be, batch3, batchf,
      cW0, cb0, cW1, cb1, cW2, cb2)


def kernel(x, edge_index, batch,
           W0, b0, g0, be0, W1, b1, g1, be1, W2, b2, g2, be2,
           cW0, cb0, cW1, cb1, cW2, cb2):
    del b0, b1, b2
    src = edge_index[0]
    dst = edge_index[1]
    dst32 = dst.reshape(_NC * _NS, -1, _CHUNK)
    eidx4 = jnp.concatenate([src.reshape(_NS, -1, _SB, _CHUNK),
                             dst.reshape(_NS, -1, _SB, _CHUNK)], axis=2)
    zeros_blk = jnp.zeros((_RPT, 128), F32)
    ones_chunk = jnp.ones((_CHUNK, 128), F32)
    batch3 = batch.reshape(_NB, 1, _BR)
    batchf = batch.astype(F32).reshape(_N, 1)
    g0r, be0r = g0.reshape(1, -1), be0.reshape(1, -1)
    g1r, be1r = g1.reshape(1, -1), be1.reshape(1, -1)
    g2r, be2r = g2.reshape(1, -1), be2.reshape(1, -1)
    cb0r, cb1r = cb0.reshape(1, -1), cb1.reshape(1, -1)
    c_real = cW2.shape[1]
    c_pad = 256 - c_real
    cW2p = jnp.concatenate([cW2, jnp.zeros((cW2.shape[0], c_pad), F32)], 1)
    cb2r = jnp.concatenate([cb2, jnp.full((c_pad,), -1e30, F32)],
                           0).reshape(1, -1)

    degparts = _sc_degree(dst32, zeros_blk, ones_chunk)

    uq, selfh = _tc_prep0(x, W0, degparts)
    S = _sc_spmm(uq, eidx4, zeros_blk)
    pre, ps, pq = _tc_stats(S, selfh, degparts)

    uq, selfh = _tc_prep(pre, ps, pq, g0r, be0r, W1, degparts)
    S = _sc_spmm(uq, eidx4, zeros_blk)
    pre, ps, pq = _tc_stats(S, selfh, degparts)

    uq, selfh = _tc_prep(pre, ps, pq, g1r, be1r, W2, degparts)
    S = _sc_spmm(uq, eidx4, zeros_blk)
    pre, ps, pq = _tc_stats(S, selfh, degparts)

    out = _tc_final(pre, ps, pq, g2r, be2r, batch3, batchf,
                    cW0, cb0r, cW1, cb1r, cW2p, cb2r)
    return out[:, :c_real]


# speedup vs baseline: 11.2211x; 1.0293x over previous
import functools

import jax
import jax.numpy as jnp
from jax import lax
from jax.experimental import pallas as pl
from jax.experimental.pallas import tpu as pltpu
from jax.experimental.pallas import tpu_sc as plsc

F32 = jnp.float32

_NC = 2
_NS = 16
_CHUNK = 80

_N = 10000
_RPT = 624
_RTAIL = _N - _RPT * _NS
_E = 320000
_G = 64
_NB = 5
_BR = _N // _NB


def _init_acc(zeros_hbm, acc, s):
    pltpu.sync_copy(zeros_hbm, acc.at[pl.ds(s * _RPT, _RPT)])

    @pl.when(s == _NS - 1)
    def _():
        pltpu.sync_copy(zeros_hbm.at[pl.ds(0, _RTAIL)],
                        acc.at[pl.ds(_RPT * _NS, _RTAIL)])


def _write_out(acc, out_hbm, c, s):
    r0 = s * _RPT
    pltpu.sync_copy(acc.at[pl.ds(r0, _RPT)],
                    out_hbm.at[c, pl.ds(r0, _RPT)])

    @pl.when(s == _NS - 1)
    def _():
        pltpu.sync_copy(acc.at[pl.ds(_RPT * _NS, _RTAIL)],
                        out_hbm.at[c, pl.ds(_RPT * _NS, _RTAIL)])


def _sc_degree(dst32, zeros_deg, ones_chunk):
    nchunks = dst32.shape[1]
    mesh = plsc.VectorSubcoreMesh(core_axis_name="c", subcore_axis_name="s")

    @functools.partial(
        pl.kernel,
        out_type=jax.ShapeDtypeStruct((_NC, _N, 128), F32),
        mesh=mesh,
        scratch_types=[
            pltpu.VMEM((nchunks, _CHUNK), jnp.int32),
            pltpu.VMEM((_CHUNK, 128), F32),
            pltpu.VMEM_SHARED((_N, 128), F32),
            pltpu.SemaphoreType.DMA,
            pltpu.SemaphoreType.DMA,
        ],
    )
    def k(dst_hbm, zeros_hbm, ones_hbm, out_hbm, didx, ones_v, acc,
          sem0, sem1):
        c = lax.axis_index("c")
        s = lax.axis_index("s")
        w = c * _NS + s
        _init_acc(zeros_hbm, acc, s)
        pltpu.sync_copy(ones_hbm, ones_v)
        pltpu.sync_copy(dst_hbm.at[w], didx)
        plsc.subcore_barrier()

        @pl.loop(0, nchunks // 2)
        def _(j):
            h0 = pltpu.async_copy(ones_v, acc.at[didx.at[2 * j]], sem0,
                                  add=True)
            h1 = pltpu.async_copy(ones_v, acc.at[didx.at[2 * j + 1]], sem1,
                                  add=True)
            h0.wait()
            h1.wait()

        if nchunks % 2:
            pltpu.sync_copy(ones_v, acc.at[didx.at[nchunks - 1]], add=True)

        plsc.subcore_barrier()
        _write_out(acc, out_hbm, c, s)

    return k(dst32, zeros_deg, ones_chunk)


_SB = 10


def _sc_spmm(uq, eidx4, zeros_blk):
    nsb = _E // (_NS * _SB * _CHUNK)
    mesh = plsc.VectorSubcoreMesh(core_axis_name="c", subcore_axis_name="s")

    @functools.partial(
        pl.kernel,
        out_type=jax.ShapeDtypeStruct((_NC, _N, 128), F32),
        mesh=mesh,
        scratch_types=[
            pltpu.VMEM((2 * _SB, _CHUNK), jnp.int32),
            pltpu.VMEM((_CHUNK, 128), F32),
            pltpu.VMEM((_CHUNK, 128), F32),
            pltpu.VMEM((_CHUNK, 128), F32),
            pltpu.VMEM((_CHUNK, 128), F32),
            pltpu.VMEM_SHARED((_N, 128), F32),
            pltpu.SemaphoreType.DMA,
            pltpu.SemaphoreType.DMA,
            pltpu.SemaphoreType.DMA,
            pltpu.SemaphoreType.DMA,
        ],
    )
    def k(ua_hbm, ub_hbm, ei_hbm, zeros_hbm, out_hbm, idxb,
          rows0, rows1, rows2, rows3, acc, gsem0, gsem1, ssem0, ssem1):
        c = lax.axis_index("c")
        s = lax.axis_index("s")
        rows = (rows0, rows1, rows2, rows3)
        gsems = (gsem0, gsem1)
        ssems = (ssem0, ssem1)
        _init_acc(zeros_hbm, acc, s)
        plsc.subcore_barrier()

        def accumulate(table):
            @pl.loop(0, nsb)
            def _(b):
                pltpu.sync_copy(ei_hbm.at[s, b], idxb)
                gw = {0: pltpu.async_copy(table.at[idxb.at[0]], rows[0],
                                          gsems[0]),
                      1: pltpu.async_copy(table.at[idxb.at[1]], rows[1],
                                          gsems[1])}
                sh = {}
                for j in range(_SB):
                    gw[j].wait()
                    if j >= 2:
                        sh[j - 2].wait()
                    sh[j] = pltpu.async_copy(rows[j % 4],
                                             acc.at[idxb.at[_SB + j]],
                                             ssems[j % 2], add=True)
                    if j + 2 < _SB:
                        gw[j + 2] = pltpu.async_copy(
                            table.at[idxb.at[j + 2]], rows[(j + 2) % 4],
                            gsems[j % 2])
                sh[_SB - 2].wait()
                sh[_SB - 1].wait()

        for cval in (0, 1):
            @pl.when(c == cval)
            def _():
                accumulate((ua_hbm, ub_hbm)[cval])

        plsc.subcore_barrier()
        _write_out(acc, out_hbm, c, s)

    return k(uq[0], uq[1], eidx4, zeros_blk)


def _deg_from_parts(dp):
    deg = dp[0, :, 0:1] + dp[1, :, 0:1] + 1.0
    return deg


def _tc_prep0(x, W0, degparts):
    def body(x_ref, w_ref, dp_ref, u0_ref, u1_ref, self_ref):
        deg = _deg_from_parts(dp_ref[...])
        dinv = lax.rsqrt(deg)
        hw = jnp.dot(x_ref[...], w_ref[...],
                     preferred_element_type=F32,
                     precision=lax.Precision.HIGHEST)
        u = hw * dinv
        u0_ref[...] = u[:, 0:128]
        u1_ref[...] = u[:, 128:256]
        self_ref[...] = hw * (1.0 / deg)

    h = W0.shape[1]
    uspec = pl.BlockSpec((_BR, 128), lambda i: (i, 0))
    ushape = jax.ShapeDtypeStruct((_N, 128), F32)
    out = pl.pallas_call(
        body,
        grid=(_NB,),
        in_specs=[
            pl.BlockSpec((_BR, x.shape[1]), lambda i: (i, 0)),
            pl.BlockSpec(W0.shape, lambda i: (0, 0)),
            pl.BlockSpec((2, _BR, 128), lambda i: (0, i, 0)),
        ],
        out_specs=[uspec, uspec,
                   pl.BlockSpec((_BR, h), lambda i: (i, 0))],
        out_shape=[ushape, ushape,
                   jax.ShapeDtypeStruct((_N, h), F32)],
    )(x, W0, degparts)
    return out[:2], out[2]


def _tc_stats(S, selfh, degparts):
    h = selfh.shape[1]

    def body(s_ref, self_ref, dp_ref, pre_ref, ps_ref, pq_ref):
        i = pl.program_id(0)
        deg = _deg_from_parts(dp_ref[...])
        dinv = lax.rsqrt(deg)
        s2 = s_ref[...]
        scat = jnp.concatenate([s2[0], s2[1]], axis=1)
        pre = dinv * scat + self_ref[...]
        pre_ref[...] = pre

        @pl.when(i == 0)
        def _():
            ps_ref[...] = jnp.zeros((8, h), F32)
            pq_ref[...] = jnp.zeros((8, h), F32)

        ps_ref[...] += pre.reshape(_BR // 8, 8, h).sum(axis=0)
        pq_ref[...] += (pre * pre).reshape(_BR // 8, 8, h).sum(axis=0)

    return pl.pallas_call(
        body,
        grid=(_NB,),
        in_specs=[
            pl.BlockSpec((2, _BR, 128), lambda i: (0, i, 0)),
            pl.BlockSpec((_BR, h), lambda i: (i, 0)),
            pl.BlockSpec((2, _BR, 128), lambda i: (0, i, 0)),
        ],
        out_specs=[
            pl.BlockSpec((_BR, h), lambda i: (i, 0)),
            pl.BlockSpec((8, h), lambda i: (0, 0)),
            pl.BlockSpec((8, h), lambda i: (0, 0)),
        ],
        out_shape=[
            jax.ShapeDtypeStruct((_N, h), F32),
            jax.ShapeDtypeStruct((8, h), F32),
            jax.ShapeDtypeStruct((8, h), F32),
        ],
    )(S, selfh, degparts)


def _tc_prep(pre, psum, psumsq, g, be, W, degparts):
    h_in = pre.shape[1]
    h_out = W.shape[1]

    def body(pre_ref, ps_ref, pq_ref, g_ref, be_ref, w_ref, dp_ref,
             u0_ref, u1_ref, self_ref):
        mean = jnp.sum(ps_ref[...], axis=0, keepdims=True) / _N
        var = jnp.sum(pq_ref[...], axis=0, keepdims=True) / _N - mean * mean
        inv = lax.rsqrt(var + 1e-5)
        hrelu = jnp.maximum(
            (pre_ref[...] - mean) * inv * g_ref[...] + be_ref[...], 0.0)
        deg = _deg_from_parts(dp_ref[...])
        dinv = lax.rsqrt(deg)
        hw = jnp.dot(hrelu, w_ref[...],
                     preferred_element_type=F32,
                     precision=lax.Precision.HIGHEST)
        u = hw * dinv
        u0_ref[...] = u[:, 0:128]
        u1_ref[...] = u[:, 128:256]
        self_ref[...] = hw * (1.0 / deg)

    uspec = pl.BlockSpec((_BR, 128), lambda i: (i, 0))
    ushape = jax.ShapeDtypeStruct((_N, 128), F32)
    out = pl.pallas_call(
        body,
        grid=(_NB,),
        in_specs=[
            pl.BlockSpec((_BR, h_in), lambda i: (i, 0)),
            pl.BlockSpec((8, h_in), lambda i: (0, 0)),
            pl.BlockSpec((8, h_in), lambda i: (0, 0)),
            pl.BlockSpec((1, h_in), lambda i: (0, 0)),
            pl.BlockSpec((1, h_in), lambda i: (0, 0)),
            pl.BlockSpec((h_in, h_out), lambda i: (0, 0)),
            pl.BlockSpec((2, _BR, 128), lambda i: (0, i, 0)),
        ],
        out_specs=[uspec, uspec,
                   pl.BlockSpec((_BR, h_out), lambda i: (i, 0))],
        out_shape=[ushape, ushape,
                   jax.ShapeDtypeStruct((_N, h_out), F32)],
    )(pre, psum, psumsq, g, be, W, degparts)
    return out[:2], out[2]


def _tc_final(pre, psum, psumsq, g, be, batch3, batchf,
              cW0, cb0, cW1, cb1, cW2, cb2):
    h = pre.shape[1]
    h1 = cW1.shape[1]
    c_out = cW2.shape[1]
    neg_inf = float("-inf")

    def body(pre_ref, ps_ref, pq_ref, g_ref, be_ref, br_ref, bc_ref,
             w0_ref, b0_ref, w1_ref, b1_ref, w2_ref, b2_ref,
             out_ref, msum_s, mmax_s, mcnt_s):
        i = pl.program_id(0)

        @pl.when(i == 0)
        def _():
            msum_s[...] = jnp.zeros((_G, h), F32)
            mmax_s[...] = jnp.full((_G, h), neg_inf, F32)
            mcnt_s[...] = jnp.zeros((_G, 128), F32)

        mean = jnp.sum(ps_ref[...], axis=0, keepdims=True) / _N
        var = jnp.sum(pq_ref[...], axis=0, keepdims=True) / _N - mean * mean
        inv = lax.rsqrt(var + 1e-5)
        hrelu = jnp.maximum(
            (pre_ref[...] - mean) * inv * g_ref[...] + be_ref[...], 0.0)

        brow = br_ref[...][0]
        oht = (lax.broadcasted_iota(jnp.int32, (_G, _BR), 0) == brow)
        oht = oht.astype(F32)
        msum_s[...] += jnp.dot(oht, hrelu, preferred_element_type=F32,
                               precision=lax.Precision.HIGHEST)
        mcnt_s[...] += jnp.dot(oht, jnp.ones((_BR, 128), F32),
                               preferred_element_type=F32,
                               precision=lax.Precision.HIGHEST)

        bcol = bc_ref[...]

        def maxbody(gid, _):
            mask = bcol == gid.astype(F32)
            rowmax = jnp.max(jnp.where(mask, hrelu, neg_inf), axis=0,
                             keepdims=True)
            cur = mmax_s[pl.ds(gid, 1), :]
            mmax_s[pl.ds(gid, 1), :] = jnp.maximum(cur, rowmax)
            return 0

        lax.fori_loop(0, _G, maxbody, 0)

        @pl.when(i == _NB - 1)
        def _():
            cnt = mcnt_s[...][:, 0:1]
            mean_p = msum_s[...] / jnp.maximum(cnt, 1.0)
            mx = jnp.where(cnt > 0.0, mmax_s[...], 0.0)
            z = jnp.concatenate([mean_p, mx], axis=1)
            z1 = jnp.maximum(jnp.dot(z, w0_ref[...],
                                     preferred_element_type=F32,
                                     precision=lax.Precision.HIGHEST)
                             + b0_ref[...], 0.0)
            z2 = jnp.maximum(jnp.dot(z1, w1_ref[...],
                                     preferred_element_type=F32,
                                     precision=lax.Precision.HIGHEST)
                             + b1_ref[...], 0.0)
            lg = jnp.dot(z2, w2_ref[...], preferred_element_type=F32,
                         precision=lax.Precision.HIGHEST) + b2_ref[...]
            m = jnp.max(lg, axis=1, keepdims=True)
            lse = jnp.log(jnp.sum(jnp.exp(lg - m), axis=1,
                                  keepdims=True)) + m
            out_ref[...] = lg - lse

    return pl.pallas_call(
        body,
        grid=(_NB,),
        in_specs=[
            pl.BlockSpec((_BR, h), lambda i: (i, 0)),
            pl.BlockSpec((8, h), lambda i: (0, 0)),
            pl.BlockSpec((8, h), lambda i: (0, 0)),
            pl.BlockSpec((1, h), lambda i: (0, 0)),
            pl.BlockSpec((1, h), lambda i: (0, 0)),
            pl.BlockSpec((1, 1, _BR), lambda i: (i, 0, 0)),
            pl.BlockSpec((_BR, 1), lambda i: (i, 0)),
            pl.BlockSpec((2 * h, h), lambda i: (0, 0)),
            pl.BlockSpec((1, h), lambda i: (0, 0)),
            pl.BlockSpec((h, h1), lambda i: (0, 0)),
            pl.BlockSpec((1, h1), lambda i: (0, 0)),
            pl.BlockSpec((h1, c_out), lambda i: (0, 0)),
            pl.BlockSpec((1, c_out), lambda i: (0, 0)),
        ],
        out_specs=pl.BlockSpec((_G, c_out), lambda i: (0, 0)),
        out_shape=jax.ShapeDtypeStruct((_G, c_out), F32),
        scratch_shapes=[
            pltpu.VMEM((_G, h), F32),
            pltpu.VMEM((_G, h), F32),
            pltpu.VMEM((_G, 128), F32),
        ],
    )(pre, psum, psumsq, g, be, batch3, batchf,
      cW0, cb0, cW1, cb1, cW2, cb2)


def kernel(x, edge_index, batch,
           W0, b0, g0, be0, W1, b1, g1, be1, W2, b2, g2, be2,
           cW0, cb0, cW1, cb1, cW2, cb2):
    del b0, b1, b2
    src = edge_index[0]
    dst = edge_index[1]
    dst32 = dst.reshape(_NC * _NS, -1, _CHUNK)
    eidx4 = jnp.concatenate([src.reshape(_NS, -1, _SB, _CHUNK),
                             dst.reshape(_NS, -1, _SB, _CHUNK)], axis=2)
    zeros_blk = jnp.zeros((_RPT, 128), F32)
    ones_chunk = jnp.ones((_CHUNK, 128), F32)
    batch3 = batch.reshape(_NB, 1, _BR)
    batchf = batch.astype(F32).reshape(_N, 1)
    g0r, be0r = g0.reshape(1, -1), be0.reshape(1, -1)
    g1r, be1r = g1.reshape(1, -1), be1.reshape(1, -1)
    g2r, be2r = g2.reshape(1, -1), be2.reshape(1, -1)
    cb0r, cb1r = cb0.reshape(1, -1), cb1.reshape(1, -1)
    c_real = cW2.shape[1]
    c_pad = 256 - c_real
    cW2p = jnp.concatenate([cW2, jnp.zeros((cW2.shape[0], c_pad), F32)], 1)
    cb2r = jnp.concatenate([cb2, jnp.full((c_pad,), -1e30, F32)],
                           0).reshape(1, -1)

    degparts = _sc_degree(dst32, zeros_blk, ones_chunk)

    uq, selfh = _tc_prep0(x, W0, degparts)
    S = _sc_spmm(uq, eidx4, zeros_blk)
    pre, ps, pq = _tc_stats(S, selfh, degparts)

    uq, selfh = _tc_prep(pre, ps, pq, g0r, be0r, W1, degparts)
    S = _sc_spmm(uq, eidx4, zeros_blk)
    pre, ps, pq = _tc_stats(S, selfh, degparts)

    uq, selfh = _tc_prep(pre, ps, pq, g1r, be1r, W2, degparts)
    S = _sc_spmm(uq, eidx4, zeros_blk)
    pre, ps, pq = _tc_stats(S, selfh, degparts)

    out = _tc_final(pre, ps, pq, g2r, be2r, batch3, batchf,
                    cW0, cb0r, cW1, cb1r, cW2p, cb2r)
    return out[:, :c_real]
